# baseline (speedup 1.0000x reference)
import functools

import jax
import jax.numpy as jnp
from jax import lax
from jax.experimental import pallas as pl
from jax.experimental.pallas import tpu as pltpu
from jax.experimental.pallas import tpu_sc as plsc

NUM_ITEM = 1000000
HIDDEN = 64
W = 128
SEQ = 200
BATCH = 4096
PAD = 3

NC, NS, L = 2, 16, 16
NW = NC * NS
N = BATCH * SEQ
PER_W = N // NW
C = 400
G = PER_W // C
PIECES = [(o, min(128, C - o)) for o in range(0, C, 128)]
NREP = C // SEQ


NBLK = NUM_ITEM // W
NTAIL = NUM_ITEM - NBLK * W


def _prep_body(tblt_hbm, tail_hbm, tblj_hbm,
               in_v0, out_v0, in_v1, out_v1, t64_v, sem0, sem1, semo):
    wid = lax.axis_index("s") * NC + lax.axis_index("c")
    lanes = jax.lax.iota(jnp.int32, L)
    nbase = NBLK // NW
    extra = NBLK % NW
    nblk = nbase + jnp.where(wid < extra, 1, 0)
    b0 = wid * nbase + jnp.minimum(wid, extra)

    def transpose_block(in_v, out_v):
        for j in range(HIDDEN):
            for lb in range(W // L):
                v = in_v[j, pl.ds(lb * L, L)]
                plsc.store_scatter(
                    out_v, [lanes + lb * L, jnp.full((L,), j, jnp.int32)], v)

    def fetch(k, in_v, sem):
        b = b0 + k
        return pltpu.async_copy(tblt_hbm.at[:, pl.ds(b * W, W)], in_v, sem)

    def wait_fetch(in_v, sem):
        pltpu.make_async_copy(tblt_hbm.at[:, pl.ds(0, W)], in_v, sem).wait()

    @pl.when(nblk > 0)
    def _go():
        fetch(0, in_v0, sem0)

        @pl.loop(0, nblk, step=2)
        def _blk(k):
            wait_fetch(in_v0, sem0)

            @pl.when(k + 1 < nblk)
            def _():
                fetch(k + 1, in_v1, sem1)

            transpose_block(in_v0, out_v0)
            pltpu.async_copy(out_v0, tblj_hbm.at[pl.ds((b0 + k) * W, W)],
                             semo).wait()

            @pl.when(k + 1 < nblk)
            def _():
                wait_fetch(in_v1, sem1)

                @pl.when(k + 2 < nblk)
                def _():
                    fetch(k + 2, in_v0, sem0)

                transpose_block(in_v1, out_v1)
                pltpu.async_copy(out_v1,
                                 tblj_hbm.at[pl.ds((b0 + k + 1) * W, W)],
                                 semo).wait()

    @pl.when(wid == 0)
    def _tail():
        pltpu.sync_copy(tail_hbm, t64_v)
        for r in range(NTAIL):
            for c in range(HIDDEN // L):
                out_v1[r, pl.ds(c * L, L)] = t64_v[r, pl.ds(c * L, L)]
        pltpu.sync_copy(out_v1.at[pl.ds(0, NTAIL)],
                        tblj_hbm.at[pl.ds(NBLK * W, NTAIL)])


@jax.jit
def _sc_prep(tblt, tail64):
    return pl.kernel(
        _prep_body,
        out_type=jax.ShapeDtypeStruct((NUM_ITEM, W), jnp.float32),
        mesh=plsc.VectorSubcoreMesh(
            core_axis_name="c", subcore_axis_name="s",
            num_cores=NC, num_subcores=NS),
        compiler_params=pltpu.CompilerParams(use_tc_tiling_on_sc=True,
                                             needs_layout_passes=False),
        scratch_types=(
            [pltpu.VMEM((HIDDEN, W), jnp.float32),
             pltpu.VMEM((W, W), jnp.float32)] * 2
            + [pltpu.VMEM((NTAIL, HIDDEN), jnp.float32)]
            + [pltpu.SemaphoreType.DMA] * 3
        ),
    )(tblt, tail64)


def _body(x_hbm, tbl_hbm, pos_hbm, out_hbm, mask_hbm,
          idx_v0, mask_v0, padl_v0, dest_v0,
          idx_v1, mask_v1, padl_v1, dest_v1,
          pos_v, sem_a0, sem_o0, sem_a1, sem_o1):
    wid = lax.axis_index("s") * NC + lax.axis_index("c")
    w0 = wid * PER_W
    lanes = jax.lax.iota(jnp.int32, L)

    pltpu.sync_copy(pos_hbm, pos_v)

    bufs = [(idx_v0, mask_v0, padl_v0, dest_v0, sem_a0, sem_o0),
            (idx_v1, mask_v1, padl_v1, dest_v1, sem_a1, sem_o1)]

    def drain_out(b):
        _, mask_v, _, dest_v, _, sem_o = bufs[b]
        pltpu.make_async_copy(dest_v.at[pl.ds(0, C)],
                              out_hbm.at[pl.ds(0, C)], sem_o).wait()
        pltpu.make_async_copy(mask_v, mask_hbm.at[pl.ds(0, C)], sem_o).wait()

    def stage1(g, b):
        idx_v, mask_v, padl_v, dest_v, sem_a, _ = bufs[b]
        base = w0 + g * C
        pltpu.sync_copy(x_hbm.at[pl.ds(base, C)], idx_v)
        cnt = jnp.int32(0)
        for j in range(C // L):
            sl = pl.ds(j * L, L)
            pad = idx_v[sl] == PAD
            padi = jnp.where(pad, 1, 0)
            mask_v[sl] = padi
            cum = plsc.cumsum(padi)
            tgt = jnp.where(pad, cnt + cum - 1, C + L)
            plsc.store_scatter(padl_v, [tgt], lanes + (j * L))
            cnt = cnt + jnp.max(cum)
        padl_v[pl.ds(cnt, L)] = jnp.full((L,), C, jnp.int32)
        for s in range(SEQ):
            for c in range(HIDDEN // L):
                v = pos_v[pl.ds(s * HIDDEN + c * L, L)]
                for rep in range(NREP):
                    dest_v[s + rep * SEQ, pl.ds(c * L, L)] = v
        descs = [
            pltpu.async_copy(tbl_hbm.at[idx_v.at[pl.ds(o, sz)]],
                             dest_v.at[pl.ds(o, sz)], sem_a, add=True)
            for o, sz in PIECES
        ]
        return descs, cnt

    def finish(g, b, descs, cnt):
        _, mask_v, padl_v, dest_v, _, sem_o = bufs[b]
        for d in descs:
            d.wait()

        zeros = jnp.zeros((L,), jnp.float32)

        @pl.loop(0, (cnt + L - 1) // L)
        def _fix(t):
            rows = padl_v[pl.ds(t * L, L)]
            for k in range(HIDDEN):
                plsc.store_scatter(
                    dest_v, [rows, jnp.full((L,), k, jnp.int32)], zeros)

        base = w0 + g * C
        pltpu.async_copy(dest_v.at[pl.ds(0, C)],
                         out_hbm.at[pl.ds(base, C)], sem_o)
        pltpu.async_copy(mask_v, mask_hbm.at[pl.ds(base, C)], sem_o)

    @pl.loop(0, G, step=2)
    def _chunk(g):
        @pl.when(g >= 2)
        def _():
            drain_out(0)

        da, ca = stage1(g, 0)

        @pl.when(g >= 2)
        def _():
            drain_out(1)

        db, cb = stage1(g + 1, 1)
        finish(g, 0, da, ca)
        finish(g + 1, 1, db, cb)

    drain_out(0)
    drain_out(1)


@jax.jit
def _sc_embed(xf, tblp, posf):
    return pl.kernel(
        _body,
        out_type=[
            jax.ShapeDtypeStruct((N, W), jnp.float32),
            jax.ShapeDtypeStruct((N,), jnp.int32),
        ],
        mesh=plsc.VectorSubcoreMesh(
            core_axis_name="c", subcore_axis_name="s",
            num_cores=NC, num_subcores=NS),
        compiler_params=pltpu.CompilerParams(use_tc_tiling_on_sc=True,
                                             needs_layout_passes=False),
        scratch_types=(
            [pltpu.VMEM((C,), jnp.int32),
             pltpu.VMEM((C,), jnp.int32),
             pltpu.VMEM((C + L + 1,), jnp.int32),
             pltpu.VMEM((C + 1, W), jnp.float32)] * 2
            + [pltpu.VMEM((SEQ * HIDDEN,), jnp.float32)]
            + [pltpu.SemaphoreType.DMA] * 4
        ),
    )(xf, tblp, posf)


def kernel(x, item_table, pos_table):
    xf = x.reshape(N)
    tblp = _sc_prep(item_table.T, item_table[NBLK * W:])
    posf = pos_table.reshape(SEQ * HIDDEN)
    emb, mask = _sc_embed(xf, tblp, posf)
    return (emb[:, :HIDDEN].reshape(BATCH, SEQ, HIDDEN),
            mask.reshape(BATCH, SEQ).astype(bool))


# speedup vs baseline: 1.0030x; 1.0030x over previous
import functools

import jax
import jax.numpy as jnp
from jax import lax
from jax.experimental import pallas as pl
from jax.experimental.pallas import tpu as pltpu
from jax.experimental.pallas import tpu_sc as plsc

NUM_ITEM = 1000000
HIDDEN = 64
W = 128
SEQ = 200
BATCH = 4096
PAD = 3

NC, NS, L = 2, 16, 16
NW = NC * NS
N = BATCH * SEQ
PER_W = N // NW
C = 400
G = PER_W // C
PIECES = [(o, min(128, C - o)) for o in range(0, C, 128)]
NREP = C // SEQ


NBLK = NUM_ITEM // W
NTAIL = NUM_ITEM - NBLK * W


SPB = 4
SPAN = SPB * W
NSPAN = NBLK // SPB
JB = HIDDEN // 8


def _prep_body(tblt_hbm, tail_hbm, tblj_hbm,
               iv0, iv1, iv2, iv3, iv4, iv5, iv6, iv7,
               out_v0, out_v1, t64_v, sem_i, sem_o0, sem_o1):
    in_vs = [iv0, iv1, iv2, iv3, iv4, iv5, iv6, iv7]
    wid = lax.axis_index("s") * NC + lax.axis_index("c")
    lanes = jax.lax.iota(jnp.int32, L)
    nbase = NSPAN // NW
    extra = NSPAN % NW
    nspan = nbase + jnp.where(wid < extra, 1, 0)
    s0 = wid * nbase + jnp.minimum(wid, extra)

    outs = [(out_v0, sem_o0), (out_v1, sem_o1)]

    def transpose_block(blk, out_v):
        for j in range(HIDDEN):
            for lb in range(W // L):
                v = in_vs[j // 8][j % 8, pl.ds(blk * W + lb * L, L)]
                plsc.store_scatter(
                    out_v, [lanes + lb * L, jnp.full((L,), j, jnp.int32)], v)

    @pl.loop(0, nspan)
    def _span(k):
        i0 = (s0 + k) * SPAN
        descs = [
            pltpu.async_copy(tblt_hbm.at[pl.ds(jb * 8, 8), pl.ds(i0, SPAN)],
                             in_vs[jb], sem_i)
            for jb in range(JB)
        ]
        for d in descs:
            d.wait()
        for blk in range(SPB):
            out_v, sem_o = outs[blk % 2]

            @pl.when(jnp.logical_or(k > 0, blk >= 2))
            def _():
                pltpu.make_async_copy(
                    out_v, tblj_hbm.at[pl.ds(0, W)], sem_o).wait()

            transpose_block(blk, out_v)
            pltpu.async_copy(out_v, tblj_hbm.at[pl.ds(i0 + blk * W, W)],
                             sem_o)

    for out_v, sem_o in outs:
        @pl.when(nspan > 0)
        def _():
            pltpu.make_async_copy(
                out_v, tblj_hbm.at[pl.ds(0, W)], sem_o).wait()

    @pl.when(wid == 0)
    def _tail():
        pltpu.sync_copy(tail_hbm, t64_v)
        for r in range(NTAIL):
            for c in range(HIDDEN // L):
                out_v0[r, pl.ds(c * L, L)] = t64_v[r, pl.ds(c * L, L)]
        pltpu.sync_copy(out_v0.at[pl.ds(0, NTAIL)],
                        tblj_hbm.at[pl.ds(NBLK * W, NTAIL)])


@jax.jit
def _sc_prep(tblt, tail64):
    return pl.kernel(
        _prep_body,
        out_type=jax.ShapeDtypeStruct((NUM_ITEM, W), jnp.float32),
        mesh=plsc.VectorSubcoreMesh(
            core_axis_name="c", subcore_axis_name="s",
            num_cores=NC, num_subcores=NS),
        compiler_params=pltpu.CompilerParams(use_tc_tiling_on_sc=True,
                                             needs_layout_passes=False),
        scratch_types=(
            [pltpu.VMEM((8, SPAN), jnp.float32)] * JB
            + [pltpu.VMEM((W, W), jnp.float32)] * 2
            + [pltpu.VMEM((NTAIL, HIDDEN), jnp.float32)]
            + [pltpu.SemaphoreType.DMA] * 3
        ),
    )(tblt, tail64)


def _body(x_hbm, tbl_hbm, pos_hbm, out_hbm, mask_hbm,
          idx_v0, mask_v0, padl_v0, dest_v0,
          idx_v1, mask_v1, padl_v1, dest_v1,
          pos_v, sem_a0, sem_o0, sem_a1, sem_o1):
    wid = lax.axis_index("s") * NC + lax.axis_index("c")
    w0 = wid * PER_W
    lanes = jax.lax.iota(jnp.int32, L)

    pltpu.sync_copy(pos_hbm, pos_v)

    bufs = [(idx_v0, mask_v0, padl_v0, dest_v0, sem_a0, sem_o0),
            (idx_v1, mask_v1, padl_v1, dest_v1, sem_a1, sem_o1)]

    def drain_out(b):
        _, mask_v, _, dest_v, _, sem_o = bufs[b]
        pltpu.make_async_copy(dest_v.at[pl.ds(0, C)],
                              out_hbm.at[pl.ds(0, C)], sem_o).wait()
        pltpu.make_async_copy(mask_v, mask_hbm.at[pl.ds(0, C)], sem_o).wait()

    def stage1(g, b):
        idx_v, mask_v, padl_v, dest_v, sem_a, _ = bufs[b]
        base = w0 + g * C
        pltpu.sync_copy(x_hbm.at[pl.ds(base, C)], idx_v)
        cnt = jnp.int32(0)
        for j in range(C // L):
            sl = pl.ds(j * L, L)
            pad = idx_v[sl] == PAD
            padi = jnp.where(pad, 1, 0)
            mask_v[sl] = padi
            cum = plsc.cumsum(padi)
            tgt = jnp.where(pad, cnt + cum - 1, C + L)
            plsc.store_scatter(padl_v, [tgt], lanes + (j * L))
            cnt = cnt + jnp.max(cum)
        padl_v[pl.ds(cnt, L)] = jnp.full((L,), C, jnp.int32)
        for s in range(SEQ):
            for c in range(HIDDEN // L):
                v = pos_v[pl.ds(s * HIDDEN + c * L, L)]
                for rep in range(NREP):
                    dest_v[s + rep * SEQ, pl.ds(c * L, L)] = v
        descs = [
            pltpu.async_copy(tbl_hbm.at[idx_v.at[pl.ds(o, sz)]],
                             dest_v.at[pl.ds(o, sz)], sem_a, add=True)
            for o, sz in PIECES
        ]
        return descs, cnt

    def finish(g, b, descs, cnt):
        _, mask_v, padl_v, dest_v, _, sem_o = bufs[b]
        for d in descs:
            d.wait()

        zeros = jnp.zeros((L,), jnp.float32)

        @pl.loop(0, (cnt + L - 1) // L)
        def _fix(t):
            rows = padl_v[pl.ds(t * L, L)]
            for k in range(HIDDEN):
                plsc.store_scatter(
                    dest_v, [rows, jnp.full((L,), k, jnp.int32)], zeros)

        base = w0 + g * C
        pltpu.async_copy(dest_v.at[pl.ds(0, C)],
                         out_hbm.at[pl.ds(base, C)], sem_o)
        pltpu.async_copy(mask_v, mask_hbm.at[pl.ds(base, C)], sem_o)

    @pl.loop(0, G, step=2)
    def _chunk(g):
        @pl.when(g >= 2)
        def _():
            drain_out(0)

        da, ca = stage1(g, 0)

        @pl.when(g >= 2)
        def _():
            drain_out(1)

        db, cb = stage1(g + 1, 1)
        finish(g, 0, da, ca)
        finish(g + 1, 1, db, cb)

    drain_out(0)
    drain_out(1)


@jax.jit
def _sc_embed(xf, tblp, posf):
    return pl.kernel(
        _body,
        out_type=[
            jax.ShapeDtypeStruct((N, W), jnp.float32),
            jax.ShapeDtypeStruct((N,), jnp.int32),
        ],
        mesh=plsc.VectorSubcoreMesh(
            core_axis_name="c", subcore_axis_name="s",
            num_cores=NC, num_subcores=NS),
        compiler_params=pltpu.CompilerParams(use_tc_tiling_on_sc=True,
                                             needs_layout_passes=False),
        scratch_types=(
            [pltpu.VMEM((C,), jnp.int32),
             pltpu.VMEM((C,), jnp.int32),
             pltpu.VMEM((C + L + 1,), jnp.int32),
             pltpu.VMEM((C + 1, W), jnp.float32)] * 2
            + [pltpu.VMEM((SEQ * HIDDEN,), jnp.float32)]
            + [pltpu.SemaphoreType.DMA] * 4
        ),
    )(xf, tblp, posf)


def kernel(x, item_table, pos_table):
    xf = x.reshape(N)
    tblp = _sc_prep(item_table.T, item_table[NBLK * W:])
    posf = pos_table.reshape(SEQ * HIDDEN)
    emb, mask = _sc_embed(xf, tblp, posf)
    return (emb[:, :HIDDEN].reshape(BATCH, SEQ, HIDDEN),
            mask.reshape(BATCH, SEQ).astype(bool))


# speedup vs baseline: 1.3490x; 1.3450x over previous
import functools

import jax
import jax.numpy as jnp
from jax import lax
from jax.experimental import pallas as pl
from jax.experimental.pallas import tpu as pltpu
from jax.experimental.pallas import tpu_sc as plsc

NUM_ITEM = 1000000
HIDDEN = 64
W = 128
SEQ = 200
BATCH = 4096
PAD = 3

NC, NS, L = 2, 16, 16
NW = NC * NS
N = BATCH * SEQ
PER_W = N // NW
C = 400
G = PER_W // C
PIECES = [(o, min(128, C - o)) for o in range(0, C, 128)]
NREP = C // SEQ


def _body(x_hbm, tbl_hbm, pos_hbm, out_hbm, mask_hbm,
          idx_v0, mask_v0, padl_v0, dest_v0,
          idx_v1, mask_v1, padl_v1, dest_v1,
          pos_v, sem_a0, sem_o0, sem_a1, sem_o1):
    wid = lax.axis_index("s") * NC + lax.axis_index("c")
    w0 = wid * PER_W
    lanes = jax.lax.iota(jnp.int32, L)

    pltpu.sync_copy(pos_hbm, pos_v)

    bufs = [(idx_v0, mask_v0, padl_v0, dest_v0, sem_a0, sem_o0),
            (idx_v1, mask_v1, padl_v1, dest_v1, sem_a1, sem_o1)]

    def drain_out(b):
        _, mask_v, _, dest_v, _, sem_o = bufs[b]
        pltpu.make_async_copy(dest_v.at[pl.ds(0, C)],
                              out_hbm.at[pl.ds(0, C)], sem_o).wait()
        pltpu.make_async_copy(mask_v, mask_hbm.at[pl.ds(0, C)], sem_o).wait()

    def stage1(g, b):
        idx_v, mask_v, padl_v, dest_v, sem_a, _ = bufs[b]
        base = w0 + g * C
        pltpu.sync_copy(x_hbm.at[pl.ds(base, C)], idx_v)
        cnt = jnp.int32(0)
        for j in range(C // L):
            sl = pl.ds(j * L, L)
            pad = idx_v[sl] == PAD
            padi = jnp.where(pad, 1, 0)
            mask_v[sl] = padi
            cum = plsc.cumsum(padi)
            tgt = jnp.where(pad, cnt + cum - 1, C + L)
            plsc.store_scatter(padl_v, [tgt], lanes + (j * L))
            cnt = cnt + jnp.max(cum)
        padl_v[pl.ds(cnt, L)] = jnp.full((L,), C, jnp.int32)
        for s in range(SEQ):
            for c in range(HIDDEN // L):
                v = pos_v[pl.ds(s * HIDDEN + c * L, L)]
                for rep in range(NREP):
                    dest_v[s + rep * SEQ, pl.ds(c * L, L)] = v
        descs = [
            pltpu.async_copy(tbl_hbm.at[idx_v.at[pl.ds(o, sz)]],
                             dest_v.at[pl.ds(o, sz)], sem_a, add=True)
            for o, sz in PIECES
        ]
        return descs, cnt

    def finish(g, b, descs, cnt):
        _, mask_v, padl_v, dest_v, _, sem_o = bufs[b]
        for d in descs:
            d.wait()

        zeros = jnp.zeros((L,), jnp.float32)

        @pl.loop(0, (cnt + L - 1) // L)
        def _fix(t):
            rows = padl_v[pl.ds(t * L, L)]
            for k in range(HIDDEN):
                plsc.store_scatter(
                    dest_v, [rows, jnp.full((L,), k, jnp.int32)], zeros)

        base = w0 + g * C
        pltpu.async_copy(dest_v.at[pl.ds(0, C)],
                         out_hbm.at[pl.ds(base, C)], sem_o)
        pltpu.async_copy(mask_v, mask_hbm.at[pl.ds(base, C)], sem_o)

    @pl.loop(0, G, step=2)
    def _chunk(g):
        @pl.when(g >= 2)
        def _():
            drain_out(0)

        da, ca = stage1(g, 0)

        @pl.when(g >= 2)
        def _():
            drain_out(1)

        db, cb = stage1(g + 1, 1)
        finish(g, 0, da, ca)
        finish(g + 1, 1, db, cb)

    drain_out(0)
    drain_out(1)


@jax.jit
def _sc_embed(xf, tblp, posf):
    return pl.kernel(
        _body,
        out_type=[
            jax.ShapeDtypeStruct((N, W), jnp.float32),
            jax.ShapeDtypeStruct((N,), jnp.int32),
        ],
        mesh=plsc.VectorSubcoreMesh(
            core_axis_name="c", subcore_axis_name="s",
            num_cores=NC, num_subcores=NS),
        compiler_params=pltpu.CompilerParams(use_tc_tiling_on_sc=True,
                                             needs_layout_passes=False),
        scratch_types=(
            [pltpu.VMEM((C,), jnp.int32),
             pltpu.VMEM((C,), jnp.int32),
             pltpu.VMEM((C + L + 1,), jnp.int32),
             pltpu.VMEM((C + 1, W), jnp.float32)] * 2
            + [pltpu.VMEM((SEQ * HIDDEN,), jnp.float32)]
            + [pltpu.SemaphoreType.DMA] * 4
        ),
    )(xf, tblp, posf)


def kernel(x, item_table, pos_table):
    xf = x.reshape(N)
    half = NUM_ITEM // 2
    tblp = jnp.concatenate(
        [jnp.pad(item_table[:half], ((0, 0), (0, W - HIDDEN))),
         jnp.pad(item_table[half:], ((0, 0), (0, W - HIDDEN)))], axis=0)
    posf = pos_table.reshape(SEQ * HIDDEN)
    emb, mask = _sc_embed(xf, tblp, posf)
    return (emb[:, :HIDDEN].reshape(BATCH, SEQ, HIDDEN),
            mask.reshape(BATCH, SEQ).astype(bool))


# speedup vs baseline: 1.7518x; 1.2987x over previous
import functools

import jax
import jax.numpy as jnp
from jax import lax
from jax.experimental import pallas as pl
from jax.experimental.pallas import tpu as pltpu
from jax.experimental.pallas import tpu_sc as plsc

NUM_ITEM = 1000000
HIDDEN = 64
W = 128
SEQ = 200
BATCH = 4096
PAD = 3

NC, NS, L = 2, 16, 16
NW = NC * NS
N = BATCH * SEQ
PER_W = N // NW
C = 400
G = PER_W // C
PIECES = [(o, min(128, C - o)) for o in range(0, C, 128)]
NREP = C // SEQ


def _body(x_hbm, tbl_hbm, pos_hbm, out_hbm, mask_hbm,
          idx_v0, mask_v0, padl_v0, dest_v0,
          idx_v1, mask_v1, padl_v1, dest_v1,
          pos_v, sem_a0, sem_o0, sem_a1, sem_o1):
    wid = lax.axis_index("s") * NC + lax.axis_index("c")
    w0 = wid * PER_W
    lanes = jax.lax.iota(jnp.int32, L)

    pltpu.sync_copy(pos_hbm, pos_v)

    bufs = [(idx_v0, mask_v0, padl_v0, dest_v0, sem_a0, sem_o0),
            (idx_v1, mask_v1, padl_v1, dest_v1, sem_a1, sem_o1)]

    def drain_out(b):
        _, mask_v, _, dest_v, _, sem_o = bufs[b]
        pltpu.make_async_copy(dest_v.at[pl.ds(0, C)],
                              out_hbm.at[pl.ds(0, C)], sem_o).wait()
        pltpu.make_async_copy(mask_v, mask_hbm.at[pl.ds(0, C)], sem_o).wait()

    def stage1(g, b):
        idx_v, mask_v, padl_v, dest_v, sem_a, _ = bufs[b]
        base = w0 + g * C
        pltpu.sync_copy(x_hbm.at[pl.ds(base, C)], idx_v)
        cnt = jnp.int32(0)
        for j in range(C // L):
            sl = pl.ds(j * L, L)
            pad = idx_v[sl] == PAD
            padi = jnp.where(pad, 1, 0)
            mask_v[sl] = padi
            cum = plsc.cumsum(padi)
            tgt = jnp.where(pad, cnt + cum - 1, C + L)
            plsc.store_scatter(padl_v, [tgt], lanes + (j * L))
            cnt = cnt + jnp.max(cum)
        padl_v[pl.ds(cnt, L)] = jnp.full((L,), C, jnp.int32)
        for s in range(SEQ):
            for c in range(HIDDEN // L):
                v = pos_v[pl.ds(s * HIDDEN + c * L, L)]
                for rep in range(NREP):
                    dest_v[s + rep * SEQ, pl.ds(c * L, L)] = v
        descs = [
            pltpu.async_copy(tbl_hbm.at[idx_v.at[pl.ds(o, sz)]],
                             dest_v.at[pl.ds(o, sz)], sem_a, add=True)
            for o, sz in PIECES
        ]
        return descs, cnt

    def finish(g, b, descs, cnt):
        _, mask_v, padl_v, dest_v, _, sem_o = bufs[b]
        for d in descs:
            d.wait()

        zeros = jnp.zeros((L,), jnp.float32)

        @pl.loop(0, (cnt + L - 1) // L)
        def _fix(t):
            rows = padl_v[pl.ds(t * L, L)]
            for k in range(HIDDEN):
                plsc.store_scatter(
                    dest_v, [rows, jnp.full((L,), k, jnp.int32)], zeros)

        base = w0 + g * C
        pltpu.async_copy(dest_v.at[pl.ds(0, C)],
                         out_hbm.at[pl.ds(base, C)], sem_o)
        pltpu.async_copy(mask_v, mask_hbm.at[pl.ds(base, C)], sem_o)

    @pl.loop(0, G, step=2)
    def _chunk(g):
        @pl.when(g >= 2)
        def _():
            drain_out(0)

        da, ca = stage1(g, 0)

        @pl.when(g >= 2)
        def _():
            drain_out(1)

        db, cb = stage1(g + 1, 1)
        finish(g, 0, da, ca)
        finish(g + 1, 1, db, cb)

    drain_out(0)
    drain_out(1)


@jax.jit
def _sc_embed(xf, tblp, posf):
    return pl.kernel(
        _body,
        out_type=[
            jax.ShapeDtypeStruct((N, W), jnp.float32),
            jax.ShapeDtypeStruct((N,), jnp.int32),
        ],
        mesh=plsc.VectorSubcoreMesh(
            core_axis_name="c", subcore_axis_name="s",
            num_cores=NC, num_subcores=NS),
        compiler_params=pltpu.CompilerParams(use_tc_tiling_on_sc=True,
                                             needs_layout_passes=False),
        scratch_types=(
            [pltpu.VMEM((C,), jnp.int32),
             pltpu.VMEM((C,), jnp.int32),
             pltpu.VMEM((C + L + 1,), jnp.int32),
             pltpu.VMEM((C + 1, W), jnp.float32)] * 2
            + [pltpu.VMEM((SEQ * HIDDEN,), jnp.float32)]
            + [pltpu.SemaphoreType.DMA] * 4
        ),
    )(xf, tblp, posf)


def kernel(x, item_table, pos_table):
    xf = x.reshape(N)
    tblp = jnp.pad(item_table, ((0, 0), (0, W - HIDDEN)))
    posf = pos_table.reshape(SEQ * HIDDEN)
    emb, mask = _sc_embed(xf, tblp, posf)
    return (emb[:, :HIDDEN].reshape(BATCH, SEQ, HIDDEN),
            mask.reshape(BATCH, SEQ).astype(bool))
